# baseline (speedup 1.0000x reference)
import functools

import jax
import jax.numpy as jnp
from jax import lax
from jax.experimental import pallas as pl
from jax.experimental.pallas import tpu as pltpu
from jax.experimental.pallas import tpu_sc as plsc

N = 10000
NPAD = 10240
H = 64
VH = 128
NPH = NPAD // 2
D = 128
C = 10
B = 64
VOCABP = 10001

NC, NS, L = 2, 16, 16
NW = NC * NS
ROWS_PT = NPAD // NS

E = 320000
ECH = 128
NCHUNK = E // ECH
CPW = NCHUNK // NW
CREM = NCHUNK - CPW * NW

LCH = 80
LK_CH = 4
LPW = LK_CH * LCH

RB = 2048
RB2 = 1024

_mesh = plsc.VectorSubcoreMesh(
    core_axis_name="c", subcore_axis_name="s", num_cores=NC, num_subcores=NS)


def _embw_body(emb_ref, w_ref, out_ref):
    i = pl.program_id(0)
    blk = emb_ref[...]
    rows = lax.broadcasted_iota(jnp.int32, blk.shape, 0) + i * RB
    blk = jnp.where(rows == 0, 0.0, blk)
    out_ref[...] = jnp.dot(blk, w_ref[...], preferred_element_type=jnp.float32)


def _embw(embed, W1):
    grid = (pl.cdiv(VOCABP, RB),)
    return pl.pallas_call(
        _embw_body,
        grid=grid,
        in_specs=[
            pl.BlockSpec((RB, D), lambda i: (i, 0)),
            pl.BlockSpec((D, H), lambda i: (0, 0)),
        ],
        out_specs=pl.BlockSpec((RB, H), lambda i: (i, 0)),
        out_shape=jax.ShapeDtypeStruct((VOCABP, H), jnp.float32),
    )(embed, W1)


@functools.partial(
    pl.kernel,
    out_type=jax.ShapeDtypeStruct((NC, NPAD), jnp.float32),
    mesh=_mesh,
    compiler_params=pltpu.CompilerParams(use_tc_tiling_on_sc=False),
    scratch_types=[
        pltpu.VMEM((CPW, ECH), jnp.int32),
        pltpu.VMEM((1, ECH), jnp.int32),
        pltpu.VMEM((ECH,), jnp.float32),
        pltpu.VMEM((ROWS_PT,), jnp.float32),
        pltpu.VMEM_SHARED((NPAD,), jnp.float32),
    ],
)
def _deg_kernel(dst_hbm, degp_hbm, idx_v, idx_x, ones_v, zbuf_v, acc_sh):
    c = lax.axis_index("c")
    s = lax.axis_index("s")
    w = c * NS + s
    for i in range(ROWS_PT // L):
        zbuf_v[pl.ds(i * L, L)] = jnp.zeros((L,), jnp.float32)
    for i in range(ECH // L):
        ones_v[pl.ds(i * L, L)] = jnp.ones((L,), jnp.float32)
    pltpu.sync_copy(zbuf_v, acc_sh.at[pl.ds(s * ROWS_PT, ROWS_PT)])
    pltpu.sync_copy(dst_hbm.at[pl.ds(w * CPW, CPW)], idx_v)
    pltpu.sync_copy(dst_hbm.at[pl.ds(NW * CPW + jnp.minimum(w, CREM - 1), 1)],
                    idx_x)
    plsc.subcore_barrier()
    for j in range(CPW):
        pltpu.sync_copy(ones_v, acc_sh.at[idx_v.at[j]], add=True)

    @pl.when(w < CREM)
    def _extra():
        pltpu.sync_copy(ones_v, acc_sh.at[idx_x.at[0]], add=True)

    plsc.subcore_barrier()
    pltpu.sync_copy(acc_sh.at[pl.ds(s * ROWS_PT, ROWS_PT)],
                    degp_hbm.at[c, pl.ds(s * ROWS_PT, ROWS_PT)])


def _dinv_body(degp_ref, out_ref):
    out_ref[...] = lax.rsqrt(degp_ref[0] + degp_ref[1] + 1.0)


def _dinv(degp_v):
    return pl.pallas_call(
        _dinv_body,
        grid=(1,),
        in_specs=[pl.BlockSpec((NC, NPAD // VH, VH), lambda i: (0, 0, 0))],
        out_specs=pl.BlockSpec((NPAD // VH, VH), lambda i: (0, 0)),
        out_shape=jax.ShapeDtypeStruct((NPAD // VH, VH), jnp.float32),
    )(degp_v)


@functools.partial(
    pl.kernel,
    out_type=[
        jax.ShapeDtypeStruct((NPAD, H), jnp.float32),
        jax.ShapeDtypeStruct((NPH, VH), jnp.float32),
    ],
    mesh=_mesh,
    compiler_params=pltpu.CompilerParams(use_tc_tiling_on_sc=False),
    scratch_types=[
        pltpu.VMEM((LPW,), jnp.int32),
        pltpu.VMEM((LPW, H), jnp.float32),
        pltpu.VMEM((LPW,), jnp.float32),
        pltpu.VMEM((LPW // 2, VH), jnp.float32),
        pltpu.SemaphoreType.DMA,
    ],
)
def _lookup_scale_kernel(embw_hbm, xi_hbm, dinv_hbm, g_hbm, dbc_hbm,
                         idx_v, rows_v, dinv_v, dbc_v, sem):
    c = lax.axis_index("c")
    s = lax.axis_index("s")
    w = c * NS + s
    base = w * LPW
    pltpu.sync_copy(xi_hbm.at[pl.ds(base, LPW)], idx_v)
    descs = []
    for j in range(LK_CH):
        descs.append(pltpu.async_copy(
            embw_hbm.at[idx_v.at[pl.ds(j * LCH, LCH)]],
            rows_v.at[pl.ds(j * LCH, LCH)], sem))
    pltpu.sync_copy(dinv_hbm.at[pl.ds(base, LPW)], dinv_v)
    for j in range(LK_CH):
        descs[j].wait()

    def body(k, carry):
        dv = dinv_v[pl.ds(k * L, L)]
        for j in range(L):
            sc = dv[j]
            r = k * L + j
            i = k * (L // 2) + j // 2
            half = (j % 2) * H
            for kk in range(H // L):
                rows_v[r, pl.ds(kk * L, L)] = (
                    rows_v[r, pl.ds(kk * L, L)] * sc)
                dbc_v[i, pl.ds(half + kk * L, L)] = jnp.full((L,), sc)
        return carry

    lax.fori_loop(0, LPW // L, body, 0)
    pltpu.sync_copy(rows_v, g_hbm.at[pl.ds(base, LPW)])
    pltpu.sync_copy(dbc_v, dbc_hbm.at[pl.ds(w * (LPW // 2), LPW // 2)])


def _make_scatter(hybrid):
    scratch = (
        [pltpu.VMEM((CPW, ECH), jnp.int32),
         pltpu.VMEM((CPW, ECH), jnp.int32),
         pltpu.VMEM((1, ECH), jnp.int32),
         pltpu.VMEM((1, ECH), jnp.int32)]
        + [pltpu.VMEM((ECH, H), jnp.float32) for _ in range(8)]
        + [pltpu.VMEM_SHARED((NPAD, H), jnp.float32)] * (2 if hybrid else 1)
        + [pltpu.SemaphoreType.DMA for _ in range(16)]
    )

    @functools.partial(
        pl.kernel,
        out_type=jax.ShapeDtypeStruct((NC, NPAD, H), jnp.float32),
        mesh=_mesh,
        compiler_params=pltpu.CompilerParams(use_tc_tiling_on_sc=False),
        scratch_types=scratch,
    )
    def _scatter_kernel(srcidx_hbm, dstidx_hbm, g_hbm, s_hbm,
                        si_v, di_v, si_x, di_x, *rest):
        NBUF, LA = 8, 4
        XB = 6
        bufs = rest[:NBUF]
        nsh = 2 if hybrid else 1
        acc_sh = rest[NBUF]
        g_sh = rest[NBUF + 1] if hybrid else None
        gsems = rest[NBUF + nsh:NBUF + nsh + NBUF]
        ssems = rest[NBUF + nsh + NBUF:NBUF + nsh + 2 * NBUF]
        c = lax.axis_index("c")
        s = lax.axis_index("s")
        w = c * NS + s
        r0 = s * ROWS_PT
        xch = NW * CPW + jnp.minimum(w, CREM - 1)
        pltpu.sync_copy(g_hbm.at[pl.ds(r0, ROWS_PT)],
                        acc_sh.at[pl.ds(r0, ROWS_PT)])
        if hybrid:
            pltpu.sync_copy(g_hbm.at[pl.ds(r0, ROWS_PT)],
                            g_sh.at[pl.ds(r0, ROWS_PT)])
        pltpu.sync_copy(srcidx_hbm.at[pl.ds(w * CPW, CPW)], si_v)
        pltpu.sync_copy(dstidx_hbm.at[pl.ds(w * CPW, CPW)], di_v)
        pltpu.sync_copy(srcidx_hbm.at[pl.ds(xch, 1)], si_x)
        pltpu.sync_copy(dstidx_hbm.at[pl.ds(xch, 1)], di_x)
        plsc.subcore_barrier()
        if hybrid:
            gsrc = [g_sh if (j % XB == XB - 1) else g_hbm for j in range(CPW)]
        else:
            gsrc = [g_hbm] * CPW
        gd = [None] * NBUF
        sd = [None] * NBUF
        for i in range(LA):
            gd[i] = pltpu.async_copy(gsrc[i].at[si_v.at[i]], bufs[i], gsems[i])
        for i in range(CPW):
            j2 = i + LA
            if j2 < CPW:
                b2 = j2 % NBUF
                gd[b2] = pltpu.async_copy(gsrc[j2].at[si_v.at[j2]], bufs[b2],
                                          gsems[b2])
            b = i % NBUF
            gd[b].wait()
            if i >= 1:
                sd[(i - 1) % NBUF].wait()
            sd[b] = pltpu.async_copy(bufs[b], acc_sh.at[di_v.at[i]], ssems[b],
                                     add=True)
        sd[(CPW - 1) % NBUF].wait()

        @pl.when(w < CREM)
        def _extra():
            pltpu.async_copy(g_hbm.at[si_x.at[0]], bufs[0], gsems[0]).wait()
            pltpu.sync_copy(bufs[0], acc_sh.at[di_x.at[0]], add=True)

        plsc.subcore_barrier()
        pltpu.sync_copy(acc_sh.at[pl.ds(r0, ROWS_PT)],
                        s_hbm.at[c, pl.ds(r0, ROWS_PT), :])

    return _scatter_kernel


_scatter_hybrid = _make_scatter(True)
_scatter_plain = _make_scatter(False)


def _mid_body(s_ref, g_ref, dbc_ref, b_ref, w2_ref, out_ref):
    t = s_ref[0] + s_ref[1] - g_ref[...]
    dbc = dbc_ref[...]
    h = jnp.maximum(dbc * t + b_ref[...], 0.0)
    out_ref[...] = dbc * jnp.dot(h, w2_ref[...],
                                 preferred_element_type=jnp.float32)


def _mid(S1v, g1v, dbc, b1c, W2blk):
    grid = (NPH // RB2,)
    return pl.pallas_call(
        _mid_body,
        grid=grid,
        in_specs=[
            pl.BlockSpec((NC, RB2, VH), lambda i: (0, i, 0)),
            pl.BlockSpec((RB2, VH), lambda i: (i, 0)),
            pl.BlockSpec((RB2, VH), lambda i: (i, 0)),
            pl.BlockSpec((1, VH), lambda i: (0, 0)),
            pl.BlockSpec((VH, VH), lambda i: (0, 0)),
        ],
        out_specs=pl.BlockSpec((RB2, VH), lambda i: (i, 0)),
        out_shape=jax.ShapeDtypeStruct((NPH, VH), jnp.float32),
    )(S1v, g1v, dbc, b1c, W2blk)


def _pool_body(s_ref, g_ref, dbc_ref, b_ref, be_ref, bo_ref, wl_ref, bl_ref,
               out_ref, pool_acc, cnt_acc):
    k = pl.program_id(0)
    t = s_ref[0] + s_ref[1] - g_ref[...]
    h2 = jnp.maximum(dbc_ref[...] * t + b_ref[...], 0.0)
    be = be_ref[...]
    bo = bo_ref[...]
    gid = lax.broadcasted_iota(jnp.int32, (B, RB2), 0)
    me = (gid == be).astype(jnp.float32)
    mo = (gid == bo).astype(jnp.float32)

    @pl.when(k == 0)
    def _init():
        pool_acc[...] = jnp.zeros_like(pool_acc)
        cnt_acc[...] = jnp.zeros_like(cnt_acc)

    pool_acc[...] += (
        jnp.dot(me, h2[:, :H], preferred_element_type=jnp.float32)
        + jnp.dot(mo, h2[:, H:], preferred_element_type=jnp.float32))
    cnt_acc[...] += (jnp.sum(me, axis=1, keepdims=True)
                     + jnp.sum(mo, axis=1, keepdims=True))

    @pl.when(k == pl.num_programs(0) - 1)
    def _fin():
        pooled = pool_acc[...] / jnp.maximum(cnt_acc[...], 1.0)
        out_ref[...] = (jnp.dot(pooled, wl_ref[...],
                                preferred_element_type=jnp.float32)
                        + bl_ref[...])


def _pool(S2v, g2v, dbc, b2c, be2, bo2, Wlin, blr):
    grid = (NPH // RB2,)
    return pl.pallas_call(
        _pool_body,
        grid=grid,
        in_specs=[
            pl.BlockSpec((NC, RB2, VH), lambda i: (0, i, 0)),
            pl.BlockSpec((RB2, VH), lambda i: (i, 0)),
            pl.BlockSpec((RB2, VH), lambda i: (i, 0)),
            pl.BlockSpec((1, VH), lambda i: (0, 0)),
            pl.BlockSpec((1, RB2), lambda i: (0, i)),
            pl.BlockSpec((1, RB2), lambda i: (0, i)),
            pl.BlockSpec((H, C), lambda i: (0, 0)),
            pl.BlockSpec((1, C), lambda i: (0, 0)),
        ],
        out_specs=pl.BlockSpec((B, C), lambda i: (0, 0)),
        out_shape=jax.ShapeDtypeStruct((B, C), jnp.float32),
        scratch_shapes=[
            pltpu.VMEM((B, H), jnp.float32),
            pltpu.VMEM((B, 1), jnp.float32),
        ],
    )(S2v, g2v, dbc, b2c, be2, bo2, Wlin, blr)


def kernel(x, edge_index, batch, embed, W1, b1, W2, b2, Wlin, blin):
    xi = x[:, 0].astype(jnp.int32)
    src_p = edge_index[0].astype(jnp.int32).reshape(NCHUNK, ECH)
    dst_p = edge_index[1].astype(jnp.int32).reshape(NCHUNK, ECH)
    xi_p = jnp.concatenate(
        [xi, jnp.zeros((NPAD - N,), dtype=jnp.int32)])
    batch_p = jnp.pad(batch.astype(jnp.int32), (0, NPAD - N),
                      constant_values=-1)
    be2 = batch_p[0::2].reshape(1, NPH)
    bo2 = batch_p[1::2].reshape(1, NPH)
    b1c = jnp.concatenate([b1, b1]).reshape(1, VH)
    b2c = jnp.concatenate([b2, b2]).reshape(1, VH)
    zden = jnp.zeros((H, H), jnp.float32)
    W2blk = jnp.concatenate(
        [jnp.concatenate([W2, zden], axis=1),
         jnp.concatenate([zden, W2], axis=1)], axis=0)
    blr = blin.reshape(1, C)

    embw = _embw(embed, W1)
    degp = _deg_kernel(dst_p)
    dinvv = _dinv(degp.reshape(NC, NPAD // VH, VH))
    g1, dbc = _lookup_scale_kernel(embw, xi_p, dinvv.reshape(NPAD))
    s1 = _scatter_plain(src_p, dst_p, g1)
    g2v = _mid(s1.reshape(NC, NPH, VH), g1.reshape(NPH, VH), dbc, b1c, W2blk)
    s2 = _scatter_plain(src_p, dst_p, g2v.reshape(NPAD, H))
    return _pool(s2.reshape(NC, NPH, VH), g2v, dbc, b2c, be2, bo2, Wlin, blr)


# speedup vs baseline: 1.0018x; 1.0018x over previous
import functools

import jax
import jax.numpy as jnp
from jax import lax
from jax.experimental import pallas as pl
from jax.experimental.pallas import tpu as pltpu
from jax.experimental.pallas import tpu_sc as plsc

N = 10000
NPAD = 10240
H = 64
VH = 128
NPH = NPAD // 2
D = 128
C = 10
B = 64
VOCABP = 10001

NC, NS, L = 2, 16, 16
NW = NC * NS
ROWS_PT = NPAD // NS

E = 320000
ECH = 128
NCHUNK = E // ECH
CPW = NCHUNK // NW
CREM = NCHUNK - CPW * NW

LCH = 80
LK_CH = 4
LPW = LK_CH * LCH

RB = 2048
RB2 = 1024

_mesh = plsc.VectorSubcoreMesh(
    core_axis_name="c", subcore_axis_name="s", num_cores=NC, num_subcores=NS)


def _embw_body(emb_ref, w_ref, out_ref):
    i = pl.program_id(0)
    blk = emb_ref[...]
    rows = lax.broadcasted_iota(jnp.int32, blk.shape, 0) + i * RB
    blk = jnp.where(rows == 0, 0.0, blk)
    out_ref[...] = jnp.dot(blk, w_ref[...], preferred_element_type=jnp.float32)


def _embw(embed, W1):
    grid = (pl.cdiv(VOCABP, RB),)
    return pl.pallas_call(
        _embw_body,
        grid=grid,
        in_specs=[
            pl.BlockSpec((RB, D), lambda i: (i, 0)),
            pl.BlockSpec((D, H), lambda i: (0, 0)),
        ],
        out_specs=pl.BlockSpec((RB, H), lambda i: (i, 0)),
        out_shape=jax.ShapeDtypeStruct((VOCABP, H), jnp.float32),
    )(embed, W1)


@functools.partial(
    pl.kernel,
    out_type=jax.ShapeDtypeStruct((NC, NPAD), jnp.float32),
    mesh=_mesh,
    compiler_params=pltpu.CompilerParams(use_tc_tiling_on_sc=False),
    scratch_types=[
        pltpu.VMEM((CPW, ECH), jnp.int32),
        pltpu.VMEM((1, ECH), jnp.int32),
        pltpu.VMEM((ECH,), jnp.float32),
        pltpu.VMEM((ROWS_PT,), jnp.float32),
        pltpu.VMEM_SHARED((NPAD,), jnp.float32),
    ],
)
def _deg_kernel(dst_hbm, degp_hbm, idx_v, idx_x, ones_v, zbuf_v, acc_sh):
    c = lax.axis_index("c")
    s = lax.axis_index("s")
    w = c * NS + s
    for i in range(ROWS_PT // L):
        zbuf_v[pl.ds(i * L, L)] = jnp.zeros((L,), jnp.float32)
    for i in range(ECH // L):
        ones_v[pl.ds(i * L, L)] = jnp.ones((L,), jnp.float32)
    pltpu.sync_copy(zbuf_v, acc_sh.at[pl.ds(s * ROWS_PT, ROWS_PT)])
    pltpu.sync_copy(dst_hbm.at[pl.ds(w * CPW, CPW)], idx_v)
    pltpu.sync_copy(dst_hbm.at[pl.ds(NW * CPW + jnp.minimum(w, CREM - 1), 1)],
                    idx_x)
    plsc.subcore_barrier()
    for j in range(CPW):
        pltpu.sync_copy(ones_v, acc_sh.at[idx_v.at[j]], add=True)

    @pl.when(w < CREM)
    def _extra():
        pltpu.sync_copy(ones_v, acc_sh.at[idx_x.at[0]], add=True)

    plsc.subcore_barrier()
    pltpu.sync_copy(acc_sh.at[pl.ds(s * ROWS_PT, ROWS_PT)],
                    degp_hbm.at[c, pl.ds(s * ROWS_PT, ROWS_PT)])


def _dinv_body(degp_ref, out_ref):
    out_ref[...] = lax.rsqrt(degp_ref[0] + degp_ref[1] + 1.0)


def _dinv(degp_v):
    return pl.pallas_call(
        _dinv_body,
        grid=(1,),
        in_specs=[pl.BlockSpec((NC, NPAD // VH, VH), lambda i: (0, 0, 0))],
        out_specs=pl.BlockSpec((NPAD // VH, VH), lambda i: (0, 0)),
        out_shape=jax.ShapeDtypeStruct((NPAD // VH, VH), jnp.float32),
    )(degp_v)


@functools.partial(
    pl.kernel,
    out_type=[
        jax.ShapeDtypeStruct((NPAD, H), jnp.float32),
        jax.ShapeDtypeStruct((NPH, VH), jnp.float32),
    ],
    mesh=_mesh,
    compiler_params=pltpu.CompilerParams(use_tc_tiling_on_sc=False),
    scratch_types=[
        pltpu.VMEM((LK_CH, LCH), jnp.int32),
        pltpu.VMEM((LPW, H), jnp.float32),
        pltpu.VMEM((LPW,), jnp.float32),
        pltpu.VMEM((LPW // 2, VH), jnp.float32),
        pltpu.SemaphoreType.DMA,
    ],
)
def _lookup_scale_kernel(embw_hbm, xi_hbm, dinv_hbm, g_hbm, dbc_hbm,
                         idx_v, rows_v, dinv_v, dbc_v, sem):
    c = lax.axis_index("c")
    s = lax.axis_index("s")
    w = c * NS + s
    base = w * LPW
    pltpu.sync_copy(xi_hbm.at[w], idx_v)
    descs = []
    for j in range(LK_CH):
        descs.append(pltpu.async_copy(
            embw_hbm.at[idx_v.at[j]], rows_v.at[pl.ds(j * LCH, LCH)], sem))
    pltpu.sync_copy(dinv_hbm.at[pl.ds(base, LPW)], dinv_v)
    for j in range(LK_CH):
        descs[j].wait()

    def body(k, carry):
        dv = dinv_v[pl.ds(k * L, L)]
        for j in range(L):
            sc = dv[j]
            r = k * L + j
            i = k * (L // 2) + j // 2
            half = (j % 2) * H
            for kk in range(H // L):
                rows_v[r, pl.ds(kk * L, L)] = (
                    rows_v[r, pl.ds(kk * L, L)] * sc)
                dbc_v[i, pl.ds(half + kk * L, L)] = jnp.full((L,), sc)
        return carry

    lax.fori_loop(0, LPW // L, body, 0)
    pltpu.sync_copy(rows_v, g_hbm.at[pl.ds(base, LPW)])
    pltpu.sync_copy(dbc_v, dbc_hbm.at[pl.ds(w * (LPW // 2), LPW // 2)])


def _make_scatter(hybrid):
    scratch = (
        [pltpu.VMEM((CPW, ECH), jnp.int32),
         pltpu.VMEM((CPW, ECH), jnp.int32),
         pltpu.VMEM((1, ECH), jnp.int32),
         pltpu.VMEM((1, ECH), jnp.int32)]
        + [pltpu.VMEM((ECH, H), jnp.float32) for _ in range(8)]
        + [pltpu.VMEM_SHARED((NPAD, H), jnp.float32)] * (2 if hybrid else 1)
        + [pltpu.SemaphoreType.DMA for _ in range(16)]
    )

    @functools.partial(
        pl.kernel,
        out_type=jax.ShapeDtypeStruct((NC, NPAD, H), jnp.float32),
        mesh=_mesh,
        compiler_params=pltpu.CompilerParams(use_tc_tiling_on_sc=False),
        scratch_types=scratch,
    )
    def _scatter_kernel(srcidx_hbm, dstidx_hbm, g_hbm, s_hbm,
                        si_v, di_v, si_x, di_x, *rest):
        NBUF, LA = 8, 4
        XB = 6
        bufs = rest[:NBUF]
        nsh = 2 if hybrid else 1
        acc_sh = rest[NBUF]
        g_sh = rest[NBUF + 1] if hybrid else None
        gsems = rest[NBUF + nsh:NBUF + nsh + NBUF]
        ssems = rest[NBUF + nsh + NBUF:NBUF + nsh + 2 * NBUF]
        c = lax.axis_index("c")
        s = lax.axis_index("s")
        w = c * NS + s
        r0 = s * ROWS_PT
        xch = NW * CPW + jnp.minimum(w, CREM - 1)
        pltpu.sync_copy(g_hbm.at[pl.ds(r0, ROWS_PT)],
                        acc_sh.at[pl.ds(r0, ROWS_PT)])
        if hybrid:
            pltpu.sync_copy(g_hbm.at[pl.ds(r0, ROWS_PT)],
                            g_sh.at[pl.ds(r0, ROWS_PT)])
        pltpu.sync_copy(srcidx_hbm.at[pl.ds(w * CPW, CPW)], si_v)
        pltpu.sync_copy(dstidx_hbm.at[pl.ds(w * CPW, CPW)], di_v)
        pltpu.sync_copy(srcidx_hbm.at[pl.ds(xch, 1)], si_x)
        pltpu.sync_copy(dstidx_hbm.at[pl.ds(xch, 1)], di_x)
        plsc.subcore_barrier()
        if hybrid:
            gsrc = [g_sh if (j % XB == XB - 1) else g_hbm for j in range(CPW)]
        else:
            gsrc = [g_hbm] * CPW
        gd = [None] * NBUF
        sd = [None] * NBUF
        for i in range(LA):
            gd[i] = pltpu.async_copy(gsrc[i].at[si_v.at[i]], bufs[i], gsems[i])
        for i in range(CPW):
            j2 = i + LA
            if j2 < CPW:
                b2 = j2 % NBUF
                gd[b2] = pltpu.async_copy(gsrc[j2].at[si_v.at[j2]], bufs[b2],
                                          gsems[b2])
            b = i % NBUF
            gd[b].wait()
            if i >= 1:
                sd[(i - 1) % NBUF].wait()
            sd[b] = pltpu.async_copy(bufs[b], acc_sh.at[di_v.at[i]], ssems[b],
                                     add=True)
        sd[(CPW - 1) % NBUF].wait()

        @pl.when(w < CREM)
        def _extra():
            pltpu.async_copy(g_hbm.at[si_x.at[0]], bufs[0], gsems[0]).wait()
            pltpu.sync_copy(bufs[0], acc_sh.at[di_x.at[0]], add=True)

        plsc.subcore_barrier()
        pltpu.sync_copy(acc_sh.at[pl.ds(r0, ROWS_PT)],
                        s_hbm.at[c, pl.ds(r0, ROWS_PT), :])

    return _scatter_kernel


_scatter_hybrid = _make_scatter(True)
_scatter_plain = _make_scatter(False)


def _mid_body(s_ref, g_ref, dbc_ref, b_ref, w2_ref, out_ref):
    t = s_ref[0] + s_ref[1] - g_ref[...]
    dbc = dbc_ref[...]
    h = jnp.maximum(dbc * t + b_ref[...], 0.0)
    out_ref[...] = dbc * jnp.dot(h, w2_ref[...],
                                 preferred_element_type=jnp.float32)


def _mid(S1v, g1v, dbc, b1c, W2blk):
    grid = (NPH // RB2,)
    return pl.pallas_call(
        _mid_body,
        grid=grid,
        in_specs=[
            pl.BlockSpec((NC, RB2, VH), lambda i: (0, i, 0)),
            pl.BlockSpec((RB2, VH), lambda i: (i, 0)),
            pl.BlockSpec((RB2, VH), lambda i: (i, 0)),
            pl.BlockSpec((1, VH), lambda i: (0, 0)),
            pl.BlockSpec((VH, VH), lambda i: (0, 0)),
        ],
        out_specs=pl.BlockSpec((RB2, VH), lambda i: (i, 0)),
        out_shape=jax.ShapeDtypeStruct((NPH, VH), jnp.float32),
    )(S1v, g1v, dbc, b1c, W2blk)


def _pool_body(s_ref, g_ref, dbc_ref, b_ref, be_ref, bo_ref, wl_ref, bl_ref,
               out_ref, pool_acc, cnt_acc):
    k = pl.program_id(0)
    t = s_ref[0] + s_ref[1] - g_ref[...]
    h2 = jnp.maximum(dbc_ref[...] * t + b_ref[...], 0.0)
    be = be_ref[...]
    bo = bo_ref[...]
    gid = lax.broadcasted_iota(jnp.int32, (B, RB2), 0)
    me = (gid == be).astype(jnp.float32)
    mo = (gid == bo).astype(jnp.float32)

    @pl.when(k == 0)
    def _init():
        pool_acc[...] = jnp.zeros_like(pool_acc)
        cnt_acc[...] = jnp.zeros_like(cnt_acc)

    pool_acc[...] += (
        jnp.dot(me, h2[:, :H], preferred_element_type=jnp.float32)
        + jnp.dot(mo, h2[:, H:], preferred_element_type=jnp.float32))
    cnt_acc[...] += (jnp.sum(me, axis=1, keepdims=True)
                     + jnp.sum(mo, axis=1, keepdims=True))

    @pl.when(k == pl.num_programs(0) - 1)
    def _fin():
        pooled = pool_acc[...] / jnp.maximum(cnt_acc[...], 1.0)
        out_ref[...] = (jnp.dot(pooled, wl_ref[...],
                                preferred_element_type=jnp.float32)
                        + bl_ref[...])


def _pool(S2v, g2v, dbc, b2c, be2, bo2, Wlin, blr):
    grid = (NPH // RB2,)
    return pl.pallas_call(
        _pool_body,
        grid=grid,
        in_specs=[
            pl.BlockSpec((NC, RB2, VH), lambda i: (0, i, 0)),
            pl.BlockSpec((RB2, VH), lambda i: (i, 0)),
            pl.BlockSpec((RB2, VH), lambda i: (i, 0)),
            pl.BlockSpec((1, VH), lambda i: (0, 0)),
            pl.BlockSpec((1, RB2), lambda i: (0, i)),
            pl.BlockSpec((1, RB2), lambda i: (0, i)),
            pl.BlockSpec((H, C), lambda i: (0, 0)),
            pl.BlockSpec((1, C), lambda i: (0, 0)),
        ],
        out_specs=pl.BlockSpec((B, C), lambda i: (0, 0)),
        out_shape=jax.ShapeDtypeStruct((B, C), jnp.float32),
        scratch_shapes=[
            pltpu.VMEM((B, H), jnp.float32),
            pltpu.VMEM((B, 1), jnp.float32),
        ],
    )(S2v, g2v, dbc, b2c, be2, bo2, Wlin, blr)


def kernel(x, edge_index, batch, embed, W1, b1, W2, b2, Wlin, blin):
    xi = x[:, 0].astype(jnp.int32)
    src_p = edge_index[0].astype(jnp.int32).reshape(NCHUNK, ECH)
    dst_p = edge_index[1].astype(jnp.int32).reshape(NCHUNK, ECH)
    xi_p = jnp.concatenate(
        [xi, jnp.zeros((NPAD - N,), dtype=jnp.int32)]).reshape(NW, LK_CH, LCH)
    batch_p = jnp.pad(batch.astype(jnp.int32), (0, NPAD - N),
                      constant_values=-1)
    be2 = batch_p[0::2].reshape(1, NPH)
    bo2 = batch_p[1::2].reshape(1, NPH)
    b1c = jnp.concatenate([b1, b1]).reshape(1, VH)
    b2c = jnp.concatenate([b2, b2]).reshape(1, VH)
    zden = jnp.zeros((H, H), jnp.float32)
    W2blk = jnp.concatenate(
        [jnp.concatenate([W2, zden], axis=1),
         jnp.concatenate([zden, W2], axis=1)], axis=0)
    blr = blin.reshape(1, C)

    embw = _embw(embed, W1)
    degp = _deg_kernel(dst_p)
    dinvv = _dinv(degp.reshape(NC, NPAD // VH, VH))
    g1, dbc = _lookup_scale_kernel(embw, xi_p, dinvv.reshape(NPAD))
    s1 = _scatter_plain(src_p, dst_p, g1)
    g2v = _mid(s1.reshape(NC, NPH, VH), g1.reshape(NPH, VH), dbc, b1c, W2blk)
    s2 = _scatter_plain(src_p, dst_p, g2v.reshape(NPAD, H))
    return _pool(s2.reshape(NC, NPH, VH), g2v, dbc, b2c, be2, bo2, Wlin, blr)


# speedup vs baseline: 1.0306x; 1.0287x over previous
import functools

import jax
import jax.numpy as jnp
from jax import lax
from jax.experimental import pallas as pl
from jax.experimental.pallas import tpu as pltpu
from jax.experimental.pallas import tpu_sc as plsc

N = 10000
NPAD = 10240
H = 64
VH = 128
NPH = NPAD // 2
D = 128
C = 10
B = 64
VOCABP = 10001

NC, NS, L = 2, 16, 16
NW = NC * NS
ROWS_PT = NPAD // NS

E = 320000
ECH = 128
NCHUNK = E // ECH
CPW = NCHUNK // NW
CREM = NCHUNK - CPW * NW

LCH = 80
LK_CH = 4
LPW = LK_CH * LCH

RB = 2048
RB2 = 1024

_mesh = plsc.VectorSubcoreMesh(
    core_axis_name="c", subcore_axis_name="s", num_cores=NC, num_subcores=NS)


def _embw_body(emb_ref, w_ref, out_ref):
    i = pl.program_id(0)
    blk = emb_ref[...]
    rows = lax.broadcasted_iota(jnp.int32, blk.shape, 0) + i * RB
    blk = jnp.where(rows == 0, 0.0, blk)
    out_ref[...] = jnp.dot(blk, w_ref[...], preferred_element_type=jnp.float32)


def _embw(embed, W1):
    grid = (pl.cdiv(VOCABP, RB),)
    return pl.pallas_call(
        _embw_body,
        grid=grid,
        in_specs=[
            pl.BlockSpec((RB, D), lambda i: (i, 0)),
            pl.BlockSpec((D, H), lambda i: (0, 0)),
        ],
        out_specs=pl.BlockSpec((RB, H), lambda i: (i, 0)),
        out_shape=jax.ShapeDtypeStruct((VOCABP, H), jnp.float32),
    )(embed, W1)


@functools.partial(
    pl.kernel,
    out_type=jax.ShapeDtypeStruct((NC, NPAD), jnp.float32),
    mesh=_mesh,
    compiler_params=pltpu.CompilerParams(use_tc_tiling_on_sc=False),
    scratch_types=[
        pltpu.VMEM((CPW, ECH), jnp.int32),
        pltpu.VMEM((1, ECH), jnp.int32),
        pltpu.VMEM((ECH,), jnp.float32),
        pltpu.VMEM((ROWS_PT,), jnp.float32),
        pltpu.VMEM_SHARED((NPAD,), jnp.float32),
    ],
)
def _deg_kernel(dst_hbm, degp_hbm, idx_v, idx_x, ones_v, zbuf_v, acc_sh):
    c = lax.axis_index("c")
    s = lax.axis_index("s")
    w = c * NS + s
    for i in range(ROWS_PT // L):
        zbuf_v[pl.ds(i * L, L)] = jnp.zeros((L,), jnp.float32)
    for i in range(ECH // L):
        ones_v[pl.ds(i * L, L)] = jnp.ones((L,), jnp.float32)
    pltpu.sync_copy(zbuf_v, acc_sh.at[pl.ds(s * ROWS_PT, ROWS_PT)])
    pltpu.sync_copy(dst_hbm.at[pl.ds(w * CPW, CPW)], idx_v)
    pltpu.sync_copy(dst_hbm.at[pl.ds(NW * CPW + jnp.minimum(w, CREM - 1), 1)],
                    idx_x)
    plsc.subcore_barrier()
    for j in range(CPW):
        pltpu.sync_copy(ones_v, acc_sh.at[idx_v.at[j]], add=True)

    @pl.when(w < CREM)
    def _extra():
        pltpu.sync_copy(ones_v, acc_sh.at[idx_x.at[0]], add=True)

    plsc.subcore_barrier()
    pltpu.sync_copy(acc_sh.at[pl.ds(s * ROWS_PT, ROWS_PT)],
                    degp_hbm.at[c, pl.ds(s * ROWS_PT, ROWS_PT)])


def _dinv_body(degp_ref, out_ref):
    out_ref[...] = lax.rsqrt(degp_ref[0] + degp_ref[1] + 1.0)


def _dinv(degp_v):
    return pl.pallas_call(
        _dinv_body,
        grid=(1,),
        in_specs=[pl.BlockSpec((NC, NPAD // VH, VH), lambda i: (0, 0, 0))],
        out_specs=pl.BlockSpec((NPAD // VH, VH), lambda i: (0, 0)),
        out_shape=jax.ShapeDtypeStruct((NPAD // VH, VH), jnp.float32),
    )(degp_v)


@functools.partial(
    pl.kernel,
    out_type=[
        jax.ShapeDtypeStruct((NPAD, H), jnp.float32),
        jax.ShapeDtypeStruct((NPH, VH), jnp.float32),
    ],
    mesh=_mesh,
    compiler_params=pltpu.CompilerParams(use_tc_tiling_on_sc=False),
    scratch_types=[
        pltpu.VMEM((LK_CH, LCH), jnp.int32),
        pltpu.VMEM((LPW, H), jnp.float32),
        pltpu.VMEM((LPW,), jnp.float32),
        pltpu.VMEM((LPW // 2, VH), jnp.float32),
        pltpu.SemaphoreType.DMA,
    ],
)
def _lookup_scale_kernel(embw_hbm, xi_hbm, dinv_hbm, g_hbm, dbc_hbm,
                         idx_v, rows_v, dinv_v, dbc_v, sem):
    c = lax.axis_index("c")
    s = lax.axis_index("s")
    w = c * NS + s
    base = w * LPW
    pltpu.sync_copy(xi_hbm.at[w], idx_v)
    descs = []
    for j in range(LK_CH):
        descs.append(pltpu.async_copy(
            embw_hbm.at[idx_v.at[j]], rows_v.at[pl.ds(j * LCH, LCH)], sem))
    pltpu.sync_copy(dinv_hbm.at[pl.ds(base, LPW)], dinv_v)
    for j in range(LK_CH):
        descs[j].wait()

    def body(k, carry):
        dv = dinv_v[pl.ds(k * L, L)]
        for j in range(L):
            sc = dv[j]
            r = k * L + j
            i = k * (L // 2) + j // 2
            half = (j % 2) * H
            for kk in range(H // L):
                rows_v[r, pl.ds(kk * L, L)] = (
                    rows_v[r, pl.ds(kk * L, L)] * sc)
                dbc_v[i, pl.ds(half + kk * L, L)] = jnp.full((L,), sc)
        return carry

    lax.fori_loop(0, LPW // L, body, 0)
    pltpu.sync_copy(rows_v, g_hbm.at[pl.ds(base, LPW)])
    pltpu.sync_copy(dbc_v, dbc_hbm.at[pl.ds(w * (LPW // 2), LPW // 2)])


def _make_scatter(hybrid):
    scratch = (
        [pltpu.VMEM((CPW, ECH), jnp.int32),
         pltpu.VMEM((CPW, ECH), jnp.int32),
         pltpu.VMEM((1, ECH), jnp.int32),
         pltpu.VMEM((1, ECH), jnp.int32)]
        + [pltpu.VMEM((ECH, H), jnp.float32) for _ in range(8)]
        + [pltpu.VMEM_SHARED((NPAD, H), jnp.float32)] * (2 if hybrid else 1)
        + [pltpu.SemaphoreType.DMA for _ in range(16)]
    )

    @functools.partial(
        pl.kernel,
        out_type=jax.ShapeDtypeStruct((NC, NPAD, H), jnp.float32),
        mesh=_mesh,
        compiler_params=pltpu.CompilerParams(use_tc_tiling_on_sc=False),
        scratch_types=scratch,
    )
    def _scatter_kernel(srcidx_hbm, dstidx_hbm, g_hbm, s_hbm,
                        si_v, di_v, si_x, di_x, *rest):
        NBUF, LA = 8, 4
        XB = 6
        bufs = rest[:NBUF]
        nsh = 2 if hybrid else 1
        acc_sh = rest[NBUF]
        g_sh = rest[NBUF + 1] if hybrid else None
        gsems = rest[NBUF + nsh:NBUF + nsh + NBUF]
        ssems = rest[NBUF + nsh + NBUF:NBUF + nsh + 2 * NBUF]
        c = lax.axis_index("c")
        s = lax.axis_index("s")
        w = c * NS + s
        r0 = s * ROWS_PT
        xch = NW * CPW + jnp.minimum(w, CREM - 1)
        pltpu.sync_copy(g_hbm.at[pl.ds(r0, ROWS_PT)],
                        acc_sh.at[pl.ds(r0, ROWS_PT)])
        if hybrid:
            pltpu.sync_copy(g_hbm.at[pl.ds(r0, ROWS_PT)],
                            g_sh.at[pl.ds(r0, ROWS_PT)])
        pltpu.sync_copy(srcidx_hbm.at[pl.ds(w * CPW, CPW)], si_v)
        pltpu.sync_copy(dstidx_hbm.at[pl.ds(w * CPW, CPW)], di_v)
        pltpu.sync_copy(srcidx_hbm.at[pl.ds(xch, 1)], si_x)
        pltpu.sync_copy(dstidx_hbm.at[pl.ds(xch, 1)], di_x)
        plsc.subcore_barrier()
        if hybrid:
            gsrc = [g_sh if (j % XB == XB - 1) else g_hbm for j in range(CPW)]
        else:
            gsrc = [g_hbm] * CPW
        gd = [None] * NBUF
        sd = [None] * NBUF
        for i in range(LA):
            gd[i] = pltpu.async_copy(gsrc[i].at[si_v.at[i]], bufs[i], gsems[i])
        for i in range(CPW):
            j2 = i + LA
            if j2 < CPW:
                b2 = j2 % NBUF
                gd[b2] = pltpu.async_copy(gsrc[j2].at[si_v.at[j2]], bufs[b2],
                                          gsems[b2])
            b = i % NBUF
            gd[b].wait()
            if i >= 1:
                sd[(i - 1) % NBUF].wait()
            sd[b] = pltpu.async_copy(bufs[b], acc_sh.at[di_v.at[i]], ssems[b],
                                     add=True)
        sd[(CPW - 1) % NBUF].wait()

        @pl.when(w < CREM)
        def _extra():
            pltpu.async_copy(g_hbm.at[si_x.at[0]], bufs[0], gsems[0]).wait()
            pltpu.sync_copy(bufs[0], acc_sh.at[di_x.at[0]], add=True)

        plsc.subcore_barrier()
        pltpu.sync_copy(acc_sh.at[pl.ds(r0, ROWS_PT)],
                        s_hbm.at[c, pl.ds(r0, ROWS_PT), :])

    return _scatter_kernel


_scatter_hybrid = _make_scatter(True)
_scatter_plain = _make_scatter(False)


def _mid_body(s_ref, g_ref, dbc_ref, b_ref, w2_ref, out_ref):
    t = s_ref[0] + s_ref[1] - g_ref[...]
    dbc = dbc_ref[...]
    h = jnp.maximum(dbc * t + b_ref[...], 0.0)
    out_ref[...] = dbc * jnp.dot(h, w2_ref[...],
                                 preferred_element_type=jnp.float32)


def _mid(S1v, g1v, dbc, b1c, W2blk):
    grid = (NPH // RB2,)
    return pl.pallas_call(
        _mid_body,
        grid=grid,
        in_specs=[
            pl.BlockSpec((NC, RB2, VH), lambda i: (0, i, 0)),
            pl.BlockSpec((RB2, VH), lambda i: (i, 0)),
            pl.BlockSpec((RB2, VH), lambda i: (i, 0)),
            pl.BlockSpec((1, VH), lambda i: (0, 0)),
            pl.BlockSpec((VH, VH), lambda i: (0, 0)),
        ],
        out_specs=pl.BlockSpec((RB2, VH), lambda i: (i, 0)),
        out_shape=jax.ShapeDtypeStruct((NPH, VH), jnp.float32),
    )(S1v, g1v, dbc, b1c, W2blk)


def _pool_body(s_ref, g_ref, dbc_ref, b_ref, be_ref, bo_ref, wl_ref, bl_ref,
               out_ref, pool_acc, cnt_acc):
    k = pl.program_id(0)
    t = s_ref[0] + s_ref[1] - g_ref[...]
    h2 = jnp.maximum(dbc_ref[...] * t + b_ref[...], 0.0)
    be = be_ref[...]
    bo = bo_ref[...]
    gid = lax.broadcasted_iota(jnp.int32, (B, RB2), 0)
    me = (gid == be).astype(jnp.float32)
    mo = (gid == bo).astype(jnp.float32)

    @pl.when(k == 0)
    def _init():
        pool_acc[...] = jnp.zeros_like(pool_acc)
        cnt_acc[...] = jnp.zeros_like(cnt_acc)

    pool_acc[...] += (
        jnp.dot(me, h2[:, :H], preferred_element_type=jnp.float32)
        + jnp.dot(mo, h2[:, H:], preferred_element_type=jnp.float32))
    cnt_acc[...] += (jnp.sum(me, axis=1, keepdims=True)
                     + jnp.sum(mo, axis=1, keepdims=True))

    @pl.when(k == pl.num_programs(0) - 1)
    def _fin():
        pooled = pool_acc[...] / jnp.maximum(cnt_acc[...], 1.0)
        out_ref[...] = (jnp.dot(pooled, wl_ref[...],
                                preferred_element_type=jnp.float32)
                        + bl_ref[...])


def _pool(S2v, g2v, dbc, b2c, be2, bo2, Wlin, blr):
    grid = (NPH // RB2,)
    return pl.pallas_call(
        _pool_body,
        grid=grid,
        in_specs=[
            pl.BlockSpec((NC, RB2, VH), lambda i: (0, i, 0)),
            pl.BlockSpec((RB2, VH), lambda i: (i, 0)),
            pl.BlockSpec((RB2, VH), lambda i: (i, 0)),
            pl.BlockSpec((1, VH), lambda i: (0, 0)),
            pl.BlockSpec((1, RB2), lambda i: (0, i)),
            pl.BlockSpec((1, RB2), lambda i: (0, i)),
            pl.BlockSpec((H, C), lambda i: (0, 0)),
            pl.BlockSpec((1, C), lambda i: (0, 0)),
        ],
        out_specs=pl.BlockSpec((B, C), lambda i: (0, 0)),
        out_shape=jax.ShapeDtypeStruct((B, C), jnp.float32),
        scratch_shapes=[
            pltpu.VMEM((B, H), jnp.float32),
            pltpu.VMEM((B, 1), jnp.float32),
        ],
    )(S2v, g2v, dbc, b2c, be2, bo2, Wlin, blr)


def kernel(x, edge_index, batch, embed, W1, b1, W2, b2, Wlin, blin):
    xi = x[:, 0].astype(jnp.int32)
    src_p = edge_index[0].astype(jnp.int32).reshape(NCHUNK, ECH)
    dst_p = edge_index[1].astype(jnp.int32).reshape(NCHUNK, ECH)
    xi_p = jnp.concatenate(
        [xi, jnp.arange(NPAD - N, dtype=jnp.int32) % VOCABP]
    ).reshape(NW, LK_CH, LCH)
    batch_p = jnp.pad(batch.astype(jnp.int32), (0, NPAD - N),
                      constant_values=-1)
    be2 = batch_p[0::2].reshape(1, NPH)
    bo2 = batch_p[1::2].reshape(1, NPH)
    b1c = jnp.concatenate([b1, b1]).reshape(1, VH)
    b2c = jnp.concatenate([b2, b2]).reshape(1, VH)
    zden = jnp.zeros((H, H), jnp.float32)
    W2blk = jnp.concatenate(
        [jnp.concatenate([W2, zden], axis=1),
         jnp.concatenate([zden, W2], axis=1)], axis=0)
    blr = blin.reshape(1, C)

    embw = _embw(embed, W1)
    degp = _deg_kernel(dst_p)
    dinvv = _dinv(degp.reshape(NC, NPAD // VH, VH))
    g1, dbc = _lookup_scale_kernel(embw, xi_p, dinvv.reshape(NPAD))
    s1 = _scatter_plain(src_p, dst_p, g1)
    g2v = _mid(s1.reshape(NC, NPH, VH), g1.reshape(NPH, VH), dbc, b1c, W2blk)
    s2 = _scatter_plain(src_p, dst_p, g2v.reshape(NPAD, H))
    return _pool(s2.reshape(NC, NPH, VH), g2v, dbc, b2c, be2, bo2, Wlin, blr)
